# baseline (speedup 1.0000x reference)
import functools

import jax
import jax.numpy as jnp
from jax import lax
from jax.experimental import pallas as pl
from jax.experimental.pallas import tpu as pltpu
from jax.experimental.pallas import tpu_sc as plsc

VOCAB = 1000000
MAXLEN = 200
DIM = 64
BATCH = 4096
SEQ = 200

NW = 32
ROWS = BATCH * SEQ
RPW = ROWS // NW
CHUNK = 100
NCH = RPW // CHUNK
NCOL = DIM // 16

_mesh = plsc.VectorSubcoreMesh(core_axis_name="c", subcore_axis_name="s")


@functools.partial(
    pl.kernel,
    out_type=jax.ShapeDtypeStruct((NW, NCH, CHUNK, DIM), jnp.float32),
    mesh=_mesh,
    compiler_params=pltpu.CompilerParams(use_tc_tiling_on_sc=False),
    scratch_types=[
        pltpu.VMEM((NCH, CHUNK), jnp.int32),
        pltpu.VMEM((MAXLEN, DIM), jnp.float32),
        pltpu.VMEM((CHUNK, DIM), jnp.float32),
        pltpu.SemaphoreType.DMA,
    ],
)
def _emb_kernel(x_hbm, tok_hbm, pos_hbm, out_hbm, idx_v, pos_v, rows_v, sem):
    wid = lax.axis_index("s") * 2 + lax.axis_index("c")
    pltpu.sync_copy(x_hbm.at[wid], idx_v)
    pltpu.sync_copy(pos_hbm, pos_v)

    def chunk_body(j, carry):
        pltpu.async_copy(tok_hbm.at[idx_v.at[j]], rows_v, sem).wait()
        pbase = (j % 2) * CHUNK

        def row_body(r, c2):
            p = pbase + r
            for c in range(NCOL):
                rows_v[r, pl.ds(c * 16, 16)] = (
                    rows_v[r, pl.ds(c * 16, 16)] + pos_v[p, pl.ds(c * 16, 16)]
                )
            return c2

        lax.fori_loop(0, CHUNK, row_body, 0, unroll=2)
        pltpu.sync_copy(rows_v, out_hbm.at[wid, j])
        return carry

    lax.fori_loop(0, NCH, chunk_body, 0)


def kernel(x, tok_table, pos_table):
    x_flat = x.reshape(NW, NCH, CHUNK)
    out = _emb_kernel(x_flat, tok_table, pos_table)
    return out.reshape(BATCH, SEQ, DIM)


# speedup vs baseline: 1.0736x; 1.0736x over previous
import functools

import jax
import jax.numpy as jnp
from jax import lax
from jax.experimental import pallas as pl
from jax.experimental.pallas import tpu as pltpu
from jax.experimental.pallas import tpu_sc as plsc

VOCAB = 1000000
MAXLEN = 200
DIM = 64
BATCH = 4096
SEQ = 200

NW = 32
ROWS = BATCH * SEQ
RPW = ROWS // NW
CHUNK = 100
NCH = RPW // CHUNK
NCOL = DIM // 16

_mesh = plsc.VectorSubcoreMesh(core_axis_name="c", subcore_axis_name="s")


@functools.partial(
    pl.kernel,
    out_type=jax.ShapeDtypeStruct((NW, NCH, CHUNK, DIM), jnp.float32),
    mesh=_mesh,
    compiler_params=pltpu.CompilerParams(use_tc_tiling_on_sc=False),
    scratch_types=[
        pltpu.VMEM((NCH, CHUNK), jnp.int32),
        pltpu.VMEM((2, CHUNK, DIM), jnp.float32),
        pltpu.VMEM((CHUNK, DIM), jnp.float32),
        pltpu.SemaphoreType.DMA,
    ],
)
def _emb_kernel(x_hbm, tok_hbm, pos_hbm, out_hbm, idx_v, pos_v, rows_v, sem):
    wid = lax.axis_index("s") * 2 + lax.axis_index("c")
    pltpu.sync_copy(x_hbm.at[wid], idx_v)
    pltpu.sync_copy(pos_hbm, pos_v)

    def chunk_body(j, carry):
        pltpu.sync_copy(pos_hbm.at[j % 2], rows_v)
        pltpu.async_copy(tok_hbm.at[idx_v.at[j]], rows_v, sem, add=True).wait()
        pltpu.sync_copy(rows_v, out_hbm.at[wid, j])
        return carry

    lax.fori_loop(0, NCH, chunk_body, 0)


def kernel(x, tok_table, pos_table):
    x_flat = x.reshape(NW, NCH, CHUNK)
    pos_r = pos_table.reshape(2, CHUNK, DIM)
    out = _emb_kernel(x_flat, tok_table, pos_r)
    return out.reshape(BATCH, SEQ, DIM)


# speedup vs baseline: 1.3004x; 1.2112x over previous
import functools

import jax
import jax.numpy as jnp
from jax import lax
from jax.experimental import pallas as pl
from jax.experimental.pallas import tpu as pltpu
from jax.experimental.pallas import tpu_sc as plsc

VOCAB = 1000000
MAXLEN = 200
DIM = 64
BATCH = 4096
SEQ = 200

NW = 32
ROWS = BATCH * SEQ
RPW = ROWS // NW
CHUNK = 100
NCH = RPW // CHUNK
NCOL = DIM // 16
NBUF = 4

_mesh = plsc.VectorSubcoreMesh(core_axis_name="c", subcore_axis_name="s")


@functools.partial(
    pl.kernel,
    out_type=jax.ShapeDtypeStruct((NW, NCH, CHUNK, DIM), jnp.float32),
    mesh=_mesh,
    compiler_params=pltpu.CompilerParams(use_tc_tiling_on_sc=False),
    scratch_types=[
        pltpu.VMEM((NCH, CHUNK), jnp.int32),
        pltpu.VMEM((2, CHUNK, DIM), jnp.float32),
        pltpu.VMEM((NBUF, CHUNK, DIM), jnp.float32),
        pltpu.SemaphoreType.DMA,
        pltpu.SemaphoreType.DMA,
    ],
)
def _emb_kernel(x_hbm, tok_hbm, pos_hbm, out_hbm, idx_v, pos_v, rows_v, gsem, ssem):
    wid = lax.axis_index("s") * 2 + lax.axis_index("c")
    pltpu.sync_copy(x_hbm.at[wid], idx_v)
    pltpu.sync_copy(pos_hbm, pos_v)

    def prefill(b):
        pb = b % 2

        def row_body(r, c2):
            for c in range(NCOL):
                rows_v[b, r, pl.ds(c * 16, 16)] = pos_v[pb, r, pl.ds(c * 16, 16)]
            return c2

        lax.fori_loop(0, CHUNK, row_body, 0, unroll=2)

    def outer(i, carry):
        gathers = []
        for b in range(NBUF):
            j = i * NBUF + b

            @pl.when(i > 0)
            def _wait_scatter():
                pltpu.make_async_copy(rows_v.at[b], out_hbm.at[wid, 0], ssem).wait()

            prefill(b)
            gathers.append(
                pltpu.async_copy(tok_hbm.at[idx_v.at[j]], rows_v.at[b], gsem, add=True)
            )
        for b in range(NBUF):
            j = i * NBUF + b
            gathers[b].wait()
            pltpu.async_copy(rows_v.at[b], out_hbm.at[wid, j], ssem)
        return carry

    lax.fori_loop(0, NCH // NBUF, outer, 0)
    for b in range(NBUF):
        pltpu.make_async_copy(rows_v.at[b], out_hbm.at[wid, 0], ssem).wait()


def kernel(x, tok_table, pos_table):
    x_flat = x.reshape(NW, NCH, CHUNK)
    pos_r = pos_table.reshape(2, CHUNK, DIM)
    out = _emb_kernel(x_flat, tok_table, pos_r)
    return out.reshape(BATCH, SEQ, DIM)


# speedup vs baseline: 1.3920x; 1.0704x over previous
import functools

import jax
import jax.numpy as jnp
from jax import lax
from jax.experimental import pallas as pl
from jax.experimental.pallas import tpu as pltpu
from jax.experimental.pallas import tpu_sc as plsc

VOCAB = 1000000
MAXLEN = 200
DIM = 64
BATCH = 4096
SEQ = 200

NW = 32
BPW = BATCH // NW
NCOL = DIM // 16
NBUF = 4

_mesh = plsc.VectorSubcoreMesh(core_axis_name="c", subcore_axis_name="s")


@functools.partial(
    pl.kernel,
    out_type=jax.ShapeDtypeStruct((BATCH, SEQ, DIM), jnp.float32),
    mesh=_mesh,
    compiler_params=pltpu.CompilerParams(use_tc_tiling_on_sc=False),
    scratch_types=[
        pltpu.VMEM((BPW, SEQ), jnp.int32),
        pltpu.VMEM((SEQ, DIM), jnp.float32),
        pltpu.VMEM((NBUF, SEQ, DIM), jnp.float32),
        pltpu.SemaphoreType.DMA,
        pltpu.SemaphoreType.DMA,
    ],
)
def _emb_kernel(x_hbm, tok_hbm, pos_hbm, out_hbm, idx_v, pos_v, rows_v, gsem, ssem):
    wid = lax.axis_index("s") * 2 + lax.axis_index("c")
    b0 = wid * BPW
    pltpu.sync_copy(x_hbm.at[pl.ds(b0, BPW)], idx_v)
    pltpu.sync_copy(pos_hbm, pos_v)

    def prefill(b):
        def row_body(r, c2):
            for c in range(NCOL):
                rows_v[b, r, pl.ds(c * 16, 16)] = pos_v[r, pl.ds(c * 16, 16)]
            return c2

        lax.fori_loop(0, SEQ, row_body, 0, unroll=2)

    def outer(i, carry):
        gathers = []
        for b in range(NBUF):
            r = i * NBUF + b

            @pl.when(i > 0)
            def _wait_scatter():
                pltpu.make_async_copy(rows_v.at[b], out_hbm.at[b0], ssem).wait()

            prefill(b)
            gathers.append(
                pltpu.async_copy(tok_hbm.at[idx_v.at[r]], rows_v.at[b], gsem, add=True)
            )
        for b in range(NBUF):
            r = i * NBUF + b
            gathers[b].wait()
            pltpu.async_copy(rows_v.at[b], out_hbm.at[b0 + r], ssem)
        return carry

    lax.fori_loop(0, BPW // NBUF, outer, 0)
    for b in range(NBUF):
        pltpu.make_async_copy(rows_v.at[b], out_hbm.at[b0], ssem).wait()


def kernel(x, tok_table, pos_table):
    return _emb_kernel(x, tok_table, pos_table)
